# Initial kernel scaffold; baseline (speedup 1.0000x reference)
import jax
import jax.numpy as jnp
from jax.experimental import pallas as pl


def kernel(x, pe_weight):
    raise NotImplementedError("write your pallas kernel here")


# speedup vs baseline: 1.6749x; 1.6749x over previous
import jax
import jax.numpy as jnp
from jax.experimental import pallas as pl


def _add_pe_kernel(x_ref, pe_ref, o_ref):
    o_ref[0] = x_ref[0] + pe_ref[...]


def kernel(x, pe_weight):
    B, T, D = x.shape
    BT = 512
    grid = (T // BT, B)
    return pl.pallas_call(
        _add_pe_kernel,
        grid=grid,
        in_specs=[
            pl.BlockSpec((1, BT, D), lambda tb, b: (b, tb, 0)),
            pl.BlockSpec((BT, D), lambda tb, b: (tb, 0)),
        ],
        out_specs=pl.BlockSpec((1, BT, D), lambda tb, b: (b, tb, 0)),
        out_shape=jax.ShapeDtypeStruct((B, T, D), x.dtype),
    )(x, pe_weight)


# speedup vs baseline: 1.7381x; 1.0377x over previous
import jax
import jax.numpy as jnp
from jax.experimental import pallas as pl


def _add_pe_kernel(x_ref, pe_ref, o_ref):
    o_ref[0] = x_ref[0] + pe_ref[...]


def kernel(x, pe_weight):
    B, T, D = x.shape
    BT = 1024
    grid = (T // BT, B)
    return pl.pallas_call(
        _add_pe_kernel,
        grid=grid,
        in_specs=[
            pl.BlockSpec((1, BT, D), lambda tb, b: (b, tb, 0)),
            pl.BlockSpec((BT, D), lambda tb, b: (tb, 0)),
        ],
        out_specs=pl.BlockSpec((1, BT, D), lambda tb, b: (b, tb, 0)),
        out_shape=jax.ShapeDtypeStruct((B, T, D), x.dtype),
    )(x, pe_weight)
